# baseline (speedup 1.0000x reference)
import functools

import jax
import jax.numpy as jnp
from jax import lax
from jax.experimental import pallas as pl
from jax.experimental.pallas import tpu as pltpu
from jax.experimental.pallas import tpu_sc as plsc

N = 1_000_000
B = 16384
MEM = 64
MSG = 100
NT = 16
NW = 32
EV_T = B // NT
NG = N // 128
GPT = 245
TRASH = N
ROUNDS = 5
KUP = 32
SLAB = 64 * 128 * 4
ROWB = 64 * 4
UPB = 8 * 128 * 4
NPAD = 1000064

_mesh = plsc.VectorSubcoreMesh(core_axis_name="c", subcore_axis_name="s")


def _i16():
    return lax.iota(jnp.int32, 16)


def _splat(x):
    return jnp.full((16,), x, jnp.int32)


@functools.partial(
    pl.kernel,
    out_type=(
        jax.ShapeDtypeStruct((B * MEM,), jnp.float32),
        jax.ShapeDtypeStruct((B // 128, 128), jnp.float32),
        jax.ShapeDtypeStruct((N,), jnp.int32),
    ),
    mesh=_mesh,
    compiler_params=pltpu.CompilerParams(needs_layout_passes=False),
    scratch_types=[
        pltpu.VMEM_SHARED((N + 16,), jnp.int32),
        pltpu.VMEM((8, 128), jnp.int32),
        pltpu.VMEM((8, 128), jnp.int32),
        pltpu.VMEM((8, 128), jnp.int32),
        pltpu.VMEM((8, 128), jnp.int32),
        pltpu.VMEM((8, 128), jnp.float32),
        pltpu.VMEM((2048,), jnp.int32),
        pltpu.VMEM((128, 128), jnp.float32),
        pltpu.VMEM((128,), jnp.int32),
        pltpu.VMEM((16384,), jnp.float32),
        pltpu.VMEM((16,), jnp.int32),
        pltpu.VMEM((16,), jnp.int32),
        pltpu.SemaphoreType.DMA,
        pltpu.SemaphoreType.DMA,
        pltpu.SemaphoreType.DMA,
    ],
)
def _phase_a(memT_hbm, lu_hbm, idx_hbm, tail_hbm, srcmem_hbm, srclast_hbm, stamp_hbm,
             stamp_sh, idx2d_v, ivals2d_v, g2d_v, dest2d_v, last2d_v,
             cneg_v, slab_v, stv_v, ring_v, rb_v, nhp_v,
             sem_in, sem_ring, sem_misc):
    cid = lax.axis_index("c")
    sid = lax.axis_index("s")
    wid = cid * NT + sid
    iota = _i16()

    for j in range(128):
        cneg_v[pl.ds(j * 16, 16)] = _splat(-1)

    def _init(k, _):
        blk = sid + k * NT

        @pl.when(blk < 488)
        def _():
            pltpu.sync_copy(cneg_v, stamp_sh.at[pl.ds(blk * 2048, 2048)])
        return 0

    lax.fori_loop(0, 31, _init, 0)

    @pl.when(sid == 0)
    def _init_tail():
        def _t(k, _):
            pltpu.sync_copy(cneg_v.at[pl.ds(0, 16)],
                            stamp_sh.at[pl.ds(999424 + k * 16, 16)])
            return 0
        lax.fori_loop(0, 37, _t, 0)

    plsc.subcore_barrier()

    ebase = sid * (EV_T // 128)
    pltpu.sync_copy(idx_hbm.at[pl.ds(ebase, 8)], idx2d_v)
    for j in range(8):
        for t in range(8):
            off = sid * EV_T + j * 128 + t * 16
            ivals2d_v[j, pl.ds(t * 16, 16)] = off + iota
    for j in range(8):
        pltpu.sync_copy(ivals2d_v.at[j], stamp_sh.at[idx2d_v.at[j]])
    plsc.subcore_barrier()
    for _ in range(ROUNDS):
        for j in range(8):
            pltpu.sync_copy(stamp_sh.at[idx2d_v.at[j]], g2d_v.at[j])
        for j in range(8):
            for t in range(8):
                s16 = pl.ds(t * 16, 16)
                act = ivals2d_v[j, s16] > g2d_v[j, s16]
                dest2d_v[j, s16] = jnp.where(act, idx2d_v[j, s16], TRASH + iota)
        for j in range(8):
            pltpu.sync_copy(ivals2d_v.at[j], stamp_sh.at[dest2d_v.at[j]])
        plsc.subcore_barrier()

    pltpu.sync_copy(idx_hbm.at[pl.ds(wid * 4, 4)], idx2d_v.at[pl.ds(0, 4)])
    for j in range(4):
        pltpu.async_copy(lu_hbm.at[idx2d_v.at[j]], last2d_v.at[j],
                         sem_misc).wait()
    pltpu.sync_copy(last2d_v.at[pl.ds(0, 4)],
                    srclast_hbm.at[pl.ds(wid * 4, 4)])

    g0 = wid * GPT
    V = jnp.clip(NG - g0, 0, GPT)
    nhp_v[...] = _splat(0)

    @pl.when(V > 0)
    def _prologue():
        off0 = pl.multiple_of(g0 * 128, 128)
        pltpu.async_copy(memT_hbm.at[:, pl.ds(off0, 128)],
                         slab_v.at[pl.ds(0, 64), :], sem_in)

    def _group(gl, _):
        @pl.when(gl < V)
        def _():
            par = gl % 2
            g = g0 + gl

            @pl.when(gl + 1 < V)
            def _():
                noff = pl.multiple_of((g + 1) * 128, 128)
                pltpu.async_copy(memT_hbm.at[:, pl.ds(noff, 128)],
                                 slab_v.at[pl.ds((1 - par) * 64, 64), :],
                                 sem_in)
            coff = pl.multiple_of(g * 128, 128)
            pltpu.sync_copy(stamp_sh.at[pl.ds(coff, 128)], stv_v)
            dexp = pltpu.async_copy(stv_v, stamp_hbm.at[pl.ds(coff, 128)],
                                    sem_misc)
            def _dr(k, _):
                pltpu.make_async_copy(srcmem_hbm.at[pl.ds(0, 64)],
                                      ring_v.at[pl.ds(0, 64)], sem_ring).wait()
                return 0

            lax.fori_loop(0, nhp_v[...][0], _dr, 0)
            pltpu.make_async_copy(memT_hbm.at[:, pl.ds(0, 128)],
                                  slab_v.at[pl.ds(0, 64), :], sem_in).wait()
            nh = 0

            def _chunk(t, nhc):
                s16 = stv_v[pl.ds(t * 16, 16)]
                cnt_t = plsc.all_reduce_population_count(s16 >= 0)[0]

                @pl.when(cnt_t > 0)
                def _(par=par):
                    for l in range(16):
                        e = s16[l]

                        @pl.when(e >= 0)
                        def _(l=l):
                            p = t * 16 + l
                            slot = par * 128 + p
                            for c4 in range(4):
                                val = plsc.load_gather(
                                    slab_v,
                                    [_splat(par * 64 + c4 * 16) + iota,
                                     _splat(p)])
                                plsc.store_scatter(
                                    ring_v,
                                    [_splat(slot * 64 + c4 * 16) + iota], val)
                            pltpu.async_copy(
                                ring_v.at[pl.ds(slot * 64, 64)],
                                srcmem_hbm.at[pl.ds(e * 64, 64)], sem_ring)
                return nhc + cnt_t

            nh = lax.fori_loop(0, 8, _chunk, 0)
            nhp_v[...] = _splat(nh)
            dexp.wait()
        return 0

    lax.fori_loop(0, GPT, _group, 0)

    def _dr_last(k, _):
        pltpu.make_async_copy(srcmem_hbm.at[pl.ds(0, 64)],
                              ring_v.at[pl.ds(0, 64)], sem_ring).wait()
        return 0

    lax.fori_loop(0, nhp_v[...][0], _dr_last, 0)

    @pl.when(wid == NW - 1)
    def _tail():
        coff = NG * 128
        pltpu.async_copy(tail_hbm.at[:, :],
                         slab_v.at[pl.ds(0, 64), :], sem_in).wait()
        pltpu.sync_copy(stamp_sh.at[pl.ds(coff, 64)], stv_v.at[pl.ds(0, 64)])
        pltpu.async_copy(stv_v.at[pl.ds(0, 64)],
                         stamp_hbm.at[pl.ds(coff, 64)], sem_misc).wait()
        def _chunk_t(t, nhc):
            s16 = stv_v[pl.ds(t * 16, 16)]
            cnt_t = plsc.all_reduce_population_count(s16 >= 0)[0]

            @pl.when(cnt_t > 0)
            def _():
                for l in range(16):
                    e = s16[l]

                    @pl.when(e >= 0)
                    def _(l=l):
                        p = t * 16 + l
                        for c4 in range(4):
                            val = plsc.load_gather(
                                slab_v, [_splat(c4 * 16) + iota, _splat(p)])
                            plsc.store_scatter(
                                ring_v, [_splat(p * 64 + c4 * 16) + iota], val)
                        pltpu.async_copy(ring_v.at[pl.ds(p * 64, 64)],
                                         srcmem_hbm.at[pl.ds(e * 64, 64)],
                                         sem_ring)
            return nhc + cnt_t

        nh = lax.fori_loop(0, 4, _chunk_t, 0)

        def _dr(k, _):
            pltpu.make_async_copy(srcmem_hbm.at[pl.ds(0, 64)],
                                  ring_v.at[pl.ds(0, 64)], sem_ring).wait()
            return 0

        lax.fori_loop(0, nh, _dr, 0)


_R = 2048


def _phase_b_body(sm_ref, ef_ref, et_ref, sl_ref,
                  w1m_ref, w1e_ref, w1d_ref, b1_ref, w2_ref, b2_ref,
                  wih_ref, whh_ref, bih_ref, bhh_ref, out_ref):
    f32 = jnp.float32
    sm = sm_ref[...]
    ef = ef_ref[...]
    dt = et_ref[...] - sl_ref[...]
    x1 = (jnp.dot(sm, w1m_ref[...], preferred_element_type=f32)
          + jnp.dot(ef, w1e_ref[...], preferred_element_type=f32)
          + dt * w1d_ref[...] + b1_ref[...])
    h1 = jnp.maximum(x1, 0.0)
    msg = jnp.dot(h1, w2_ref[...], preferred_element_type=f32) + b2_ref[...]
    gi = jnp.dot(msg, wih_ref[...], preferred_element_type=f32) + bih_ref[...]
    gh = jnp.dot(sm, whh_ref[...], preferred_element_type=f32) + bhh_ref[...]
    r = jax.nn.sigmoid(gi[:, 0:MEM] + gh[:, 0:MEM])
    z = jax.nn.sigmoid(gi[:, MEM:2 * MEM] + gh[:, MEM:2 * MEM])
    n = jnp.tanh(gi[:, 2 * MEM:3 * MEM] + r * gh[:, 2 * MEM:3 * MEM])
    upd = (1.0 - z) * n + z * sm
    out_ref[...] = jnp.concatenate([upd, jnp.zeros((_R, 128 - MEM), f32)],
                                   axis=1)


def _phase_b(src_mem, ef, et2d, sl2d, w1m, w1e, w1d, b1, w2, b2,
             wih, whh, bih, bhh):
    full = lambda shape: pl.BlockSpec(shape, lambda i: (0, 0))
    return pl.pallas_call(
        _phase_b_body,
        grid=(B // _R,),
        in_specs=[
            pl.BlockSpec((_R, MEM), lambda i: (i, 0)),
            pl.BlockSpec((_R, MEM), lambda i: (i, 0)),
            pl.BlockSpec((_R, 1), lambda i: (i, 0)),
            pl.BlockSpec((_R, 1), lambda i: (i, 0)),
            full((MEM, MSG)), full((MEM, MSG)), full((1, MSG)), full((1, MSG)),
            full((MSG, MSG)), full((1, MSG)),
            full((MSG, 3 * MEM)), full((MEM, 3 * MEM)),
            full((1, 3 * MEM)), full((1, 3 * MEM)),
        ],
        out_specs=pl.BlockSpec((_R, 128), lambda i: (i, 0)),
        out_shape=jax.ShapeDtypeStruct((B, 128), jnp.float32),
    )(src_mem, ef, et2d, sl2d, w1m, w1e, w1d, b1, w2, b2, wih, whh, bih, bhh)


@functools.partial(
    pl.kernel,
    out_type=jax.ShapeDtypeStruct((MEM, NPAD), jnp.float32),
    mesh=_mesh,
    compiler_params=pltpu.CompilerParams(needs_layout_passes=False),
    scratch_types=[
        pltpu.VMEM((192, 128), jnp.float32),
        pltpu.VMEM((256,), jnp.int32),
        pltpu.VMEM((KUP * 8, 128), jnp.float32),
        pltpu.VMEM((16,), jnp.int32),
        pltpu.SemaphoreType.DMA,
        pltpu.SemaphoreType.DMA,
        pltpu.SemaphoreType.DMA,
        pltpu.SemaphoreType.DMA,
    ],
)
def _phase_c(memT_hbm, upd_hbm, stamp_hbm, tail_hbm, outT_hbm,
             slab_v, stv_v, updr_v, rb_v,
             sem_in, sem_out, sem_up, sem_st):
    cid = lax.axis_index("c")
    sid = lax.axis_index("s")
    wid = cid * NT + sid
    iota = _i16()
    g0 = wid * GPT
    V = jnp.clip(NG - g0, 0, GPT)

    @pl.when(V > 0)
    def _prologue():
        off0 = pl.multiple_of(g0 * 128, 128)
        pltpu.async_copy(memT_hbm.at[:, pl.ds(off0, 128)],
                         slab_v.at[pl.ds(0, 64), :], sem_in)
        pltpu.async_copy(stamp_hbm.at[pl.ds(off0, 128)],
                         stv_v.at[pl.ds(0, 128)], sem_st)

    def _group(gl, _):
        @pl.when((gl >= 2) & (gl - 2 < V))
        def _():
            pltpu.make_async_copy(memT_hbm.at[:, pl.ds(0, 128)],
                                  slab_v.at[pl.ds(0, 64), :], sem_out).wait()

        @pl.when(gl < V)
        def _():
            par = gl % 3
            spar = gl % 2
            g = g0 + gl

            @pl.when(gl + 1 < V)
            def _():
                noff = pl.multiple_of((g + 1) * 128, 128)
                pltpu.async_copy(memT_hbm.at[:, pl.ds(noff, 128)],
                                 slab_v.at[pl.ds(((gl + 1) % 3) * 64, 64), :],
                                 sem_in)
                pltpu.async_copy(stamp_hbm.at[pl.ds(noff, 128)],
                                 stv_v.at[pl.ds(((gl + 1) % 2) * 128, 128)],
                                 sem_st)
            pltpu.make_async_copy(stamp_hbm.at[pl.ds(0, 128)],
                                  stv_v.at[pl.ds(0, 128)], sem_st).wait()
            pltpu.make_async_copy(memT_hbm.at[:, pl.ds(0, 128)],
                                  slab_v.at[pl.ds(0, 64), :], sem_in).wait()

            sbase = spar * 128
            s16s = [stv_v[pl.ds(sbase + t * 16, 16)] for t in range(8)]
            cnts = [plsc.all_reduce_population_count(s >= 0)[0] for s in s16s]
            bases = [0]
            for t in range(8):
                bases.append(bases[t] + cnts[t])
            nh = bases[8]

            @pl.when(nh > 0)
            def _(par=par, s16s=s16s, cnts=cnts, bases=bases, nh=nh):
                def _wave(w, _):
                    wlo = w * KUP
                    whi = wlo + KUP
                    for t in range(8):
                        lc = bases[t]
                        for l in range(16):
                            e = s16s[t][l]
                            onr = e >= 0

                            @pl.when(onr & (lc >= wlo) & (lc < whi))
                            def _(e=e, lc=lc):
                                e8 = pl.multiple_of((e // 8) * 8, 8)
                                pltpu.async_copy(
                                    upd_hbm.at[pl.ds(e8, 8), :],
                                    updr_v.at[pl.ds((lc - wlo) * 8, 8), :],
                                    sem_up)
                            lc = lc + jnp.where(onr, 1, 0)
                    cw = jnp.minimum(nh, whi) - wlo

                    def _drw(k, _):
                        pltpu.make_async_copy(upd_hbm.at[pl.ds(0, 8), :],
                                              updr_v.at[pl.ds(0, 8), :],
                                              sem_up).wait()
                        return 0

                    lax.fori_loop(0, cw, _drw, 0)
                    for t in range(8):
                        lc = bases[t]
                        for l in range(16):
                            e = s16s[t][l]
                            onr = e >= 0

                            @pl.when(onr & (lc >= wlo) & (lc < whi))
                            def _(e=e, lc=lc, t=t, l=l):
                                r = (lc - wlo) * 8 + (e - (e // 8) * 8)
                                p = t * 16 + l
                                for c4 in range(4):
                                    val = plsc.load_gather(
                                        updr_v, [_splat(r), c4 * 16 + iota])
                                    plsc.store_scatter(
                                        slab_v,
                                        [_splat(par * 64 + c4 * 16) + iota,
                                         _splat(p)], val)
                            lc = lc + jnp.where(onr, 1, 0)
                    return 0

                lax.fori_loop(0, (nh + KUP - 1) // KUP, _wave, 0)

            coff = pl.multiple_of(g * 128, 128)
            pltpu.async_copy(slab_v.at[pl.ds(par * 64, 64), :],
                             outT_hbm.at[:, pl.ds(coff, 128)], sem_out)
        return 0

    lax.fori_loop(0, GPT, _group, 0)

    @pl.when(V >= GPT - 1)
    def _ep1():
        pltpu.make_async_copy(memT_hbm.at[:, pl.ds(0, 128)],
                              slab_v.at[pl.ds(0, 64), :], sem_out).wait()

    @pl.when(V >= GPT)
    def _ep2():
        pltpu.make_async_copy(memT_hbm.at[:, pl.ds(0, 128)],
                              slab_v.at[pl.ds(0, 64), :], sem_out).wait()

    @pl.when(wid == NW - 1)
    def _tail():
        coff = NG * 128
        pltpu.async_copy(tail_hbm.at[:, :],
                         slab_v.at[pl.ds(0, 64), :], sem_in).wait()
        pltpu.sync_copy(stamp_hbm.at[pl.ds(coff, 64)], stv_v.at[pl.ds(0, 64)])
        for t in range(4):
            s16 = stv_v[pl.ds(t * 16, 16)]
            for l in range(16):
                e = s16[l]

                @pl.when(e >= 0)
                def _(e=e, t=t, l=l):
                    e8 = pl.multiple_of((e // 8) * 8, 8)
                    pltpu.async_copy(upd_hbm.at[pl.ds(e8, 8), :],
                                     updr_v.at[pl.ds(0, 8), :], sem_up).wait()
                    r = e - (e // 8) * 8
                    p = t * 16 + l
                    for c4 in range(4):
                        val = plsc.load_gather(updr_v,
                                               [_splat(r), c4 * 16 + iota])
                        plsc.store_scatter(slab_v,
                                           [c4 * 16 + iota, _splat(p)], val)
        pltpu.async_copy(slab_v.at[pl.ds(0, 64), :],
                         outT_hbm.at[:, pl.ds(coff, 128)], sem_out).wait()


def kernel(source_nodes, edge_times, edge_features, memory, last_update,
           W1, b1, W2, b2, W_ih, W_hh, b_ih, b_hh):
    idx2d = source_nodes.reshape(B // 128, 128)
    memT = memory.T
    tail_in = jnp.pad(memory[NG * 128:, :].T, ((0, 0), (0, NPAD - N)))
    srcmem_lin, src_last2d, stamp = _phase_a(memT, last_update, idx2d, tail_in)
    updated = _phase_b(
        srcmem_lin.reshape(B, MEM), edge_features,
        edge_times.reshape(B, 1), src_last2d.reshape(B, 1),
        W1[:, :MEM].T, W1[:, MEM:2 * MEM].T, W1[:, 2 * MEM].reshape(1, MSG),
        b1.reshape(1, MSG), W2.T, b2.reshape(1, MSG),
        W_ih.T, W_hh.T, b_ih.reshape(1, 3 * MEM), b_hh.reshape(1, 3 * MEM),
    )
    outT = _phase_c(memT, updated, stamp, tail_in)
    return outT[:, :N].T


# speedup vs baseline: 3.5006x; 3.5006x over previous
import functools

import jax
import jax.numpy as jnp
from jax import lax
from jax.experimental import pallas as pl
from jax.experimental.pallas import tpu as pltpu
from jax.experimental.pallas import tpu_sc as plsc

N = 1_000_000
B = 16384
MEM = 64
MSG = 100
NT = 16
NW = 32
EV_T = B // NT
NG = N // 128
GPT = 245
TRASH = N
ROUNDS = 5
KUP = 16
SLAB = 64 * 128 * 4
ROWB = 64 * 4
UPB = 8 * 128 * 4
NPAD = 1000064

_mesh = plsc.VectorSubcoreMesh(core_axis_name="c", subcore_axis_name="s")


def _i16():
    return lax.iota(jnp.int32, 16)


def _splat(x):
    return jnp.full((16,), x, jnp.int32)


@functools.partial(
    pl.kernel,
    out_type=(
        jax.ShapeDtypeStruct((B * MEM,), jnp.float32),
        jax.ShapeDtypeStruct((B // 128, 128), jnp.float32),
        jax.ShapeDtypeStruct((N,), jnp.int32),
    ),
    mesh=_mesh,
    compiler_params=pltpu.CompilerParams(needs_layout_passes=False),
    scratch_types=[
        pltpu.VMEM_SHARED((N + 16,), jnp.int32),
        pltpu.VMEM((8, 128), jnp.int32),
        pltpu.VMEM((8, 128), jnp.int32),
        pltpu.VMEM((8, 128), jnp.int32),
        pltpu.VMEM((8, 128), jnp.int32),
        pltpu.VMEM((8, 128), jnp.float32),
        pltpu.VMEM((2048,), jnp.int32),
        pltpu.VMEM((128, 128), jnp.float32),
        pltpu.VMEM((128,), jnp.int32),
        pltpu.VMEM((16384,), jnp.float32),
        pltpu.VMEM((16,), jnp.int32),
        pltpu.VMEM((16,), jnp.int32),
        pltpu.SemaphoreType.DMA,
        pltpu.SemaphoreType.DMA,
        pltpu.SemaphoreType.DMA,
    ],
)
def _phase_a(memT_hbm, lu_hbm, idx_hbm, tail_hbm, srcmem_hbm, srclast_hbm, stamp_hbm,
             stamp_sh, idx2d_v, ivals2d_v, g2d_v, dest2d_v, last2d_v,
             cneg_v, slab_v, stv_v, ring_v, rb_v, nhp_v,
             sem_in, sem_ring, sem_misc):
    cid = lax.axis_index("c")
    sid = lax.axis_index("s")
    wid = cid * NT + sid
    iota = _i16()

    for j in range(128):
        cneg_v[pl.ds(j * 16, 16)] = _splat(-1)

    def _init(k, _):
        blk = sid + k * NT

        @pl.when(blk < 488)
        def _():
            pltpu.sync_copy(cneg_v, stamp_sh.at[pl.ds(blk * 2048, 2048)])
        return 0

    lax.fori_loop(0, 31, _init, 0)

    @pl.when(sid == 0)
    def _init_tail():
        def _t(k, _):
            pltpu.sync_copy(cneg_v.at[pl.ds(0, 16)],
                            stamp_sh.at[pl.ds(999424 + k * 16, 16)])
            return 0
        lax.fori_loop(0, 37, _t, 0)

    plsc.subcore_barrier()

    ebase = sid * (EV_T // 128)
    pltpu.sync_copy(idx_hbm.at[pl.ds(ebase, 8)], idx2d_v)
    for j in range(8):
        for t in range(8):
            off = sid * EV_T + j * 128 + t * 16
            ivals2d_v[j, pl.ds(t * 16, 16)] = off + iota
    for j in range(8):
        pltpu.sync_copy(ivals2d_v.at[j], stamp_sh.at[idx2d_v.at[j]])
    plsc.subcore_barrier()
    for _ in range(ROUNDS):
        for j in range(8):
            pltpu.sync_copy(stamp_sh.at[idx2d_v.at[j]], g2d_v.at[j])
        for j in range(8):
            for t in range(8):
                s16 = pl.ds(t * 16, 16)
                act = ivals2d_v[j, s16] > g2d_v[j, s16]
                dest2d_v[j, s16] = jnp.where(act, idx2d_v[j, s16], TRASH + iota)
        for j in range(8):
            pltpu.sync_copy(ivals2d_v.at[j], stamp_sh.at[dest2d_v.at[j]])
        plsc.subcore_barrier()

    pltpu.sync_copy(idx_hbm.at[pl.ds(wid * 4, 4)], idx2d_v.at[pl.ds(0, 4)])
    for j in range(4):
        pltpu.async_copy(lu_hbm.at[idx2d_v.at[j]], last2d_v.at[j],
                         sem_misc).wait()
    pltpu.sync_copy(last2d_v.at[pl.ds(0, 4)],
                    srclast_hbm.at[pl.ds(wid * 4, 4)])

    g0 = wid * GPT
    V = jnp.clip(NG - g0, 0, GPT)
    nhp_v[...] = _splat(0)

    @pl.when(V > 0)
    def _prologue():
        off0 = pl.multiple_of(g0 * 128, 128)
        pltpu.async_copy(memT_hbm.at[:, pl.ds(off0, 128)],
                         slab_v.at[pl.ds(0, 64), :], sem_in)

    def _group(gl, _):
        @pl.when(gl < V)
        def _():
            par = gl % 2
            g = g0 + gl

            @pl.when(gl + 1 < V)
            def _():
                noff = pl.multiple_of((g + 1) * 128, 128)
                pltpu.async_copy(memT_hbm.at[:, pl.ds(noff, 128)],
                                 slab_v.at[pl.ds((1 - par) * 64, 64), :],
                                 sem_in)
            coff = pl.multiple_of(g * 128, 128)
            pltpu.sync_copy(stamp_sh.at[pl.ds(coff, 128)], stv_v)
            dexp = pltpu.async_copy(stv_v, stamp_hbm.at[pl.ds(coff, 128)],
                                    sem_misc)
            def _dr(k, _):
                pltpu.make_async_copy(srcmem_hbm.at[pl.ds(0, 64)],
                                      ring_v.at[pl.ds(0, 64)], sem_ring).wait()
                return 0

            lax.fori_loop(0, nhp_v[...][0], _dr, 0)
            pltpu.make_async_copy(memT_hbm.at[:, pl.ds(0, 128)],
                                  slab_v.at[pl.ds(0, 64), :], sem_in).wait()
            nh = 0

            def _chunk(t, nhc):
                s16 = stv_v[pl.ds(t * 16, 16)]
                cnt_t = plsc.all_reduce_population_count(s16 >= 0)[0]

                @pl.when(cnt_t > 0)
                def _(par=par):
                    for l in range(16):
                        e = s16[l]

                        @pl.when(e >= 0)
                        def _(l=l):
                            p = t * 16 + l
                            slot = par * 128 + p
                            for c4 in range(4):
                                val = plsc.load_gather(
                                    slab_v,
                                    [_splat(par * 64 + c4 * 16) + iota,
                                     _splat(p)])
                                plsc.store_scatter(
                                    ring_v,
                                    [_splat(slot * 64 + c4 * 16) + iota], val)
                            pltpu.async_copy(
                                ring_v.at[pl.ds(slot * 64, 64)],
                                srcmem_hbm.at[pl.ds(e * 64, 64)], sem_ring)
                return nhc + cnt_t

            nh = lax.fori_loop(0, 8, _chunk, 0)
            nhp_v[...] = _splat(nh)
            dexp.wait()
        return 0

    lax.fori_loop(0, GPT, _group, 0)

    def _dr_last(k, _):
        pltpu.make_async_copy(srcmem_hbm.at[pl.ds(0, 64)],
                              ring_v.at[pl.ds(0, 64)], sem_ring).wait()
        return 0

    lax.fori_loop(0, nhp_v[...][0], _dr_last, 0)

    @pl.when(wid == NW - 1)
    def _tail():
        coff = NG * 128
        pltpu.async_copy(tail_hbm.at[:, :],
                         slab_v.at[pl.ds(0, 64), :], sem_in).wait()
        pltpu.sync_copy(stamp_sh.at[pl.ds(coff, 64)], stv_v.at[pl.ds(0, 64)])
        pltpu.async_copy(stv_v.at[pl.ds(0, 64)],
                         stamp_hbm.at[pl.ds(coff, 64)], sem_misc).wait()
        def _chunk_t(t, nhc):
            s16 = stv_v[pl.ds(t * 16, 16)]
            cnt_t = plsc.all_reduce_population_count(s16 >= 0)[0]

            @pl.when(cnt_t > 0)
            def _():
                for l in range(16):
                    e = s16[l]

                    @pl.when(e >= 0)
                    def _(l=l):
                        p = t * 16 + l
                        for c4 in range(4):
                            val = plsc.load_gather(
                                slab_v, [_splat(c4 * 16) + iota, _splat(p)])
                            plsc.store_scatter(
                                ring_v, [_splat(p * 64 + c4 * 16) + iota], val)
                        pltpu.async_copy(ring_v.at[pl.ds(p * 64, 64)],
                                         srcmem_hbm.at[pl.ds(e * 64, 64)],
                                         sem_ring)
            return nhc + cnt_t

        nh = lax.fori_loop(0, 4, _chunk_t, 0)

        def _dr(k, _):
            pltpu.make_async_copy(srcmem_hbm.at[pl.ds(0, 64)],
                                  ring_v.at[pl.ds(0, 64)], sem_ring).wait()
            return 0

        lax.fori_loop(0, nh, _dr, 0)


_R = 2048


def _phase_b_body(sm_ref, ef_ref, et_ref, sl_ref,
                  w1m_ref, w1e_ref, w1d_ref, b1_ref, w2_ref, b2_ref,
                  wih_ref, whh_ref, bih_ref, bhh_ref, out_ref):
    f32 = jnp.float32
    sm = sm_ref[...]
    ef = ef_ref[...]
    dt = et_ref[...] - sl_ref[...]
    x1 = (jnp.dot(sm, w1m_ref[...], preferred_element_type=f32)
          + jnp.dot(ef, w1e_ref[...], preferred_element_type=f32)
          + dt * w1d_ref[...] + b1_ref[...])
    h1 = jnp.maximum(x1, 0.0)
    msg = jnp.dot(h1, w2_ref[...], preferred_element_type=f32) + b2_ref[...]
    gi = jnp.dot(msg, wih_ref[...], preferred_element_type=f32) + bih_ref[...]
    gh = jnp.dot(sm, whh_ref[...], preferred_element_type=f32) + bhh_ref[...]
    r = jax.nn.sigmoid(gi[:, 0:MEM] + gh[:, 0:MEM])
    z = jax.nn.sigmoid(gi[:, MEM:2 * MEM] + gh[:, MEM:2 * MEM])
    n = jnp.tanh(gi[:, 2 * MEM:3 * MEM] + r * gh[:, 2 * MEM:3 * MEM])
    upd = (1.0 - z) * n + z * sm
    out_ref[...] = jnp.concatenate([upd, jnp.zeros((_R, 128 - MEM), f32)],
                                   axis=1)


def _phase_b(src_mem, ef, et2d, sl2d, w1m, w1e, w1d, b1, w2, b2,
             wih, whh, bih, bhh):
    full = lambda shape: pl.BlockSpec(shape, lambda i: (0, 0))
    return pl.pallas_call(
        _phase_b_body,
        grid=(B // _R,),
        in_specs=[
            pl.BlockSpec((_R, MEM), lambda i: (i, 0)),
            pl.BlockSpec((_R, MEM), lambda i: (i, 0)),
            pl.BlockSpec((_R, 1), lambda i: (i, 0)),
            pl.BlockSpec((_R, 1), lambda i: (i, 0)),
            full((MEM, MSG)), full((MEM, MSG)), full((1, MSG)), full((1, MSG)),
            full((MSG, MSG)), full((1, MSG)),
            full((MSG, 3 * MEM)), full((MEM, 3 * MEM)),
            full((1, 3 * MEM)), full((1, 3 * MEM)),
        ],
        out_specs=pl.BlockSpec((_R, 128), lambda i: (i, 0)),
        out_shape=jax.ShapeDtypeStruct((B, 128), jnp.float32),
    )(src_mem, ef, et2d, sl2d, w1m, w1e, w1d, b1, w2, b2, wih, whh, bih, bhh)


SG = 4
LW = 128 * SG
NSUP = (NG * 128) // LW
SPT = 62


@functools.partial(
    pl.kernel,
    out_type=jax.ShapeDtypeStruct((MEM, NPAD), jnp.float32),
    mesh=_mesh,
    compiler_params=pltpu.CompilerParams(needs_layout_passes=False),
    scratch_types=[
        pltpu.VMEM((128, LW), jnp.float32),
        pltpu.VMEM((2 * LW,), jnp.int32),
        pltpu.VMEM((KUP * 8, 128), jnp.float32),
        pltpu.SemaphoreType.DMA,
        pltpu.SemaphoreType.DMA,
        pltpu.SemaphoreType.DMA,
        pltpu.SemaphoreType.DMA,
    ],
)
def _phase_c(memT_hbm, upd_hbm, stamp_hbm, tail_hbm, outT_hbm,
             slab_v, stv_v, updr_v,
             sem_in, sem_out, sem_up, sem_st):
    cid = lax.axis_index("c")
    sid = lax.axis_index("s")
    wid = cid * NT + sid
    iota = _i16()
    s0 = wid * SPT
    V = jnp.clip(NSUP - s0, 0, SPT)

    @pl.when(V > 0)
    def _prologue():
        off0 = pl.multiple_of(s0 * LW, 128)
        pltpu.async_copy(memT_hbm.at[:, pl.ds(off0, LW)],
                         slab_v.at[pl.ds(0, 64), :], sem_in)
        pltpu.async_copy(stamp_hbm.at[pl.ds(off0, LW)],
                         stv_v.at[pl.ds(0, LW)], sem_st)

    def _super(gs, _):
        @pl.when(gs < V)
        def _():
            par = gs % 2
            g = s0 + gs

            @pl.when((gs >= 1) & (gs + 1 < V))
            def _():
                pltpu.make_async_copy(memT_hbm.at[:, pl.ds(0, LW)],
                                      slab_v.at[pl.ds(0, 64), :],
                                      sem_out).wait()

            @pl.when(gs + 1 < V)
            def _():
                noff = pl.multiple_of((g + 1) * LW, 128)
                pltpu.async_copy(memT_hbm.at[:, pl.ds(noff, LW)],
                                 slab_v.at[pl.ds((1 - par) * 64, 64), :],
                                 sem_in)
                pltpu.async_copy(stamp_hbm.at[pl.ds(noff, LW)],
                                 stv_v.at[pl.ds((1 - par) * LW, LW)], sem_st)
            pltpu.make_async_copy(stamp_hbm.at[pl.ds(0, LW)],
                                  stv_v.at[pl.ds(0, LW)], sem_st).wait()
            pltpu.make_async_copy(memT_hbm.at[:, pl.ds(0, LW)],
                                  slab_v.at[pl.ds(0, 64), :], sem_in).wait()
            sbase = par * LW

            def _cnt(t, a):
                s16 = stv_v[pl.ds(sbase + t * 16, 16)]
                return a + plsc.all_reduce_population_count(s16 >= 0)[0]

            nh = lax.fori_loop(0, LW // 16, _cnt, 0)

            @pl.when(nh > 0)
            def _(par=par, sbase=sbase, nh=nh):
                def _wave(w, _):
                    wlo = w * KUP
                    whi = wlo + KUP

                    def _chunk_issue(t, base):
                        s16 = stv_v[pl.ds(sbase + t * 16, 16)]
                        cnt = plsc.all_reduce_population_count(s16 >= 0)[0]

                        @pl.when((cnt > 0) & (base < whi)
                                 & (base + cnt > wlo))
                        def _():
                            lc = base
                            for l in range(16):
                                e = s16[l]
                                onr = e >= 0

                                @pl.when(onr & (lc >= wlo) & (lc < whi))
                                def _(e=e, lc=lc):
                                    e8 = pl.multiple_of((e // 8) * 8, 8)
                                    pltpu.async_copy(
                                        upd_hbm.at[pl.ds(e8, 8), :],
                                        updr_v.at[pl.ds((lc - wlo) * 8, 8), :],
                                        sem_up)
                                lc = lc + jnp.where(onr, 1, 0)
                        return base + cnt

                    lax.fori_loop(0, LW // 16, _chunk_issue, 0)
                    cw = jnp.minimum(nh, whi) - wlo

                    def _drw(k, _):
                        pltpu.make_async_copy(upd_hbm.at[pl.ds(0, 8), :],
                                              updr_v.at[pl.ds(0, 8), :],
                                              sem_up).wait()
                        return 0

                    lax.fori_loop(0, cw, _drw, 0)

                    def _chunk_apply(t, base):
                        s16 = stv_v[pl.ds(sbase + t * 16, 16)]
                        cnt = plsc.all_reduce_population_count(s16 >= 0)[0]

                        @pl.when((cnt > 0) & (base < whi)
                                 & (base + cnt > wlo))
                        def _():
                            lc = base
                            for l in range(16):
                                e = s16[l]
                                onr = e >= 0

                                @pl.when(onr & (lc >= wlo) & (lc < whi))
                                def _(e=e, lc=lc, l=l):
                                    r = (lc - wlo) * 8 + (e - (e // 8) * 8)
                                    p = t * 16 + l
                                    for c4 in range(4):
                                        val = plsc.load_gather(
                                            updr_v,
                                            [_splat(r), c4 * 16 + iota])
                                        plsc.store_scatter(
                                            slab_v,
                                            [_splat(par * 64 + c4 * 16) + iota,
                                             _splat(p)], val)
                                lc = lc + jnp.where(onr, 1, 0)
                        return base + cnt

                    lax.fori_loop(0, LW // 16, _chunk_apply, 0)
                    return 0

                lax.fori_loop(0, (nh + KUP - 1) // KUP, _wave, 0)

            coff = pl.multiple_of(g * LW, 128)
            pltpu.async_copy(slab_v.at[pl.ds(par * 64, 64), :],
                             outT_hbm.at[:, pl.ds(coff, LW)], sem_out)
        return 0

    lax.fori_loop(0, SPT, _super, 0)

    @pl.when(V >= 2)
    def _ep1():
        pltpu.make_async_copy(memT_hbm.at[:, pl.ds(0, LW)],
                              slab_v.at[pl.ds(0, 64), :], sem_out).wait()

    @pl.when(V >= 1)
    def _ep2():
        pltpu.make_async_copy(memT_hbm.at[:, pl.ds(0, LW)],
                              slab_v.at[pl.ds(0, 64), :], sem_out).wait()

    @pl.when(wid == NW - 1)
    def _tail():
        coff = NG * 128
        pltpu.async_copy(tail_hbm.at[:, :],
                         slab_v.at[pl.ds(0, 64), pl.ds(0, 128)], sem_in).wait()
        pltpu.sync_copy(stamp_hbm.at[pl.ds(coff, 64)], stv_v.at[pl.ds(0, 64)])
        for t in range(4):
            s16 = stv_v[pl.ds(t * 16, 16)]
            for l in range(16):
                e = s16[l]

                @pl.when(e >= 0)
                def _(e=e, t=t, l=l):
                    e8 = pl.multiple_of((e // 8) * 8, 8)
                    pltpu.async_copy(upd_hbm.at[pl.ds(e8, 8), :],
                                     updr_v.at[pl.ds(0, 8), :], sem_up).wait()
                    r = e - (e // 8) * 8
                    p = t * 16 + l
                    for c4 in range(4):
                        val = plsc.load_gather(updr_v,
                                               [_splat(r), c4 * 16 + iota])
                        plsc.store_scatter(slab_v,
                                           [c4 * 16 + iota, _splat(p)], val)
        pltpu.async_copy(slab_v.at[pl.ds(0, 64), pl.ds(0, 128)],
                         outT_hbm.at[:, pl.ds(coff, 128)], sem_out).wait()


def kernel(source_nodes, edge_times, edge_features, memory, last_update,
           W1, b1, W2, b2, W_ih, W_hh, b_ih, b_hh):
    idx2d = source_nodes.reshape(B // 128, 128)
    memT = memory.T
    tail_in = jnp.pad(memory[NG * 128:, :].T, ((0, 0), (0, NPAD - N)))
    srcmem_lin, src_last2d, stamp = _phase_a(memT, last_update, idx2d, tail_in)
    updated = _phase_b(
        srcmem_lin.reshape(B, MEM), edge_features,
        edge_times.reshape(B, 1), src_last2d.reshape(B, 1),
        W1[:, :MEM].T, W1[:, MEM:2 * MEM].T, W1[:, 2 * MEM].reshape(1, MSG),
        b1.reshape(1, MSG), W2.T, b2.reshape(1, MSG),
        W_ih.T, W_hh.T, b_ih.reshape(1, 3 * MEM), b_hh.reshape(1, 3 * MEM),
    )
    outT = _phase_c(memT, updated, stamp, tail_in)
    return outT[:, :N].T


# speedup vs baseline: 3.6288x; 1.0366x over previous
import functools

import jax
import jax.numpy as jnp
from jax import lax
from jax.experimental import pallas as pl
from jax.experimental.pallas import tpu as pltpu
from jax.experimental.pallas import tpu_sc as plsc

N = 1_000_000
B = 16384
MEM = 64
MSG = 100
NT = 16
NW = 32
EV_T = B // NT
NG = N // 128
GPT = 245
TRASH = N
ROUNDS = 5
KUP = 16
SLAB = 64 * 128 * 4
ROWB = 64 * 4
UPB = 8 * 128 * 4
NPAD = 1000064

_mesh = plsc.VectorSubcoreMesh(core_axis_name="c", subcore_axis_name="s")


def _i16():
    return lax.iota(jnp.int32, 16)


def _splat(x):
    return jnp.full((16,), x, jnp.int32)


@functools.partial(
    pl.kernel,
    out_type=(
        jax.ShapeDtypeStruct((B * MEM,), jnp.float32),
        jax.ShapeDtypeStruct((B // 128, 128), jnp.float32),
        jax.ShapeDtypeStruct((N,), jnp.int32),
    ),
    mesh=_mesh,
    compiler_params=pltpu.CompilerParams(needs_layout_passes=False),
    scratch_types=[
        pltpu.VMEM_SHARED((N + 16,), jnp.int32),
        pltpu.VMEM((8, 128), jnp.int32),
        pltpu.VMEM((8, 128), jnp.int32),
        pltpu.VMEM((8, 128), jnp.int32),
        pltpu.VMEM((8, 128), jnp.int32),
        pltpu.VMEM((8, 128), jnp.float32),
        pltpu.VMEM((2048,), jnp.int32),
        pltpu.VMEM((128, 256), jnp.float32),
        pltpu.VMEM((256,), jnp.int32),
        pltpu.VMEM((16384,), jnp.float32),
        pltpu.VMEM((16,), jnp.int32),
        pltpu.VMEM((16,), jnp.int32),
        pltpu.SemaphoreType.DMA,
        pltpu.SemaphoreType.DMA,
        pltpu.SemaphoreType.DMA,
    ],
)
def _phase_a(memT_hbm, lu_hbm, idx_hbm, tail_hbm, srcmem_hbm, srclast_hbm, stamp_hbm,
             stamp_sh, idx2d_v, ivals2d_v, g2d_v, dest2d_v, last2d_v,
             cneg_v, slab_v, stv_v, ring_v, rb_v, nhp_v,
             sem_in, sem_ring, sem_misc):
    cid = lax.axis_index("c")
    sid = lax.axis_index("s")
    wid = cid * NT + sid
    iota = _i16()

    for j in range(128):
        cneg_v[pl.ds(j * 16, 16)] = _splat(-1)

    def _init(k, _):
        blk = sid + k * NT

        @pl.when(blk < 488)
        def _():
            pltpu.sync_copy(cneg_v, stamp_sh.at[pl.ds(blk * 2048, 2048)])
        return 0

    lax.fori_loop(0, 31, _init, 0)

    @pl.when(sid == 0)
    def _init_tail():
        def _t(k, _):
            pltpu.sync_copy(cneg_v.at[pl.ds(0, 16)],
                            stamp_sh.at[pl.ds(999424 + k * 16, 16)])
            return 0
        lax.fori_loop(0, 37, _t, 0)

    plsc.subcore_barrier()

    ebase = sid * (EV_T // 128)
    pltpu.sync_copy(idx_hbm.at[pl.ds(ebase, 8)], idx2d_v)
    for j in range(8):
        for t in range(8):
            off = sid * EV_T + j * 128 + t * 16
            ivals2d_v[j, pl.ds(t * 16, 16)] = off + iota
    for j in range(8):
        pltpu.sync_copy(ivals2d_v.at[j], stamp_sh.at[idx2d_v.at[j]])
    plsc.subcore_barrier()
    for _ in range(ROUNDS):
        for j in range(8):
            pltpu.sync_copy(stamp_sh.at[idx2d_v.at[j]], g2d_v.at[j])
        for j in range(8):
            for t in range(8):
                s16 = pl.ds(t * 16, 16)
                act = ivals2d_v[j, s16] > g2d_v[j, s16]
                dest2d_v[j, s16] = jnp.where(act, idx2d_v[j, s16], TRASH + iota)
        for j in range(8):
            pltpu.sync_copy(ivals2d_v.at[j], stamp_sh.at[dest2d_v.at[j]])
        plsc.subcore_barrier()

    pltpu.sync_copy(idx_hbm.at[pl.ds(wid * 4, 4)], idx2d_v.at[pl.ds(0, 4)])
    for j in range(4):
        pltpu.async_copy(lu_hbm.at[idx2d_v.at[j]], last2d_v.at[j],
                         sem_misc).wait()
    pltpu.sync_copy(last2d_v.at[pl.ds(0, 4)],
                    srclast_hbm.at[pl.ds(wid * 4, 4)])

    LWA = 256
    NSUPA = (NG * 128) // LWA
    SPTA = 123
    s0 = wid * SPTA
    V = jnp.clip(NSUPA - s0, 0, SPTA)
    nhp_v[...] = _splat(0)

    @pl.when(V > 0)
    def _prologue():
        off0 = pl.multiple_of(s0 * LWA, 128)
        pltpu.async_copy(memT_hbm.at[:, pl.ds(off0, LWA)],
                         slab_v.at[pl.ds(0, 64), :], sem_in)

    def _super(gs, _):
        @pl.when(gs < V)
        def _():
            par = gs % 2
            g = s0 + gs

            @pl.when(gs + 1 < V)
            def _():
                noff = pl.multiple_of((g + 1) * LWA, 128)
                pltpu.async_copy(memT_hbm.at[:, pl.ds(noff, LWA)],
                                 slab_v.at[pl.ds((1 - par) * 64, 64), :],
                                 sem_in)
            coff = pl.multiple_of(g * LWA, 128)
            pltpu.sync_copy(stamp_sh.at[pl.ds(coff, LWA)], stv_v)
            dexp = pltpu.async_copy(stv_v, stamp_hbm.at[pl.ds(coff, LWA)],
                                    sem_misc)

            def _dr(k, _):
                pltpu.make_async_copy(srcmem_hbm.at[pl.ds(0, 64)],
                                      ring_v.at[pl.ds(0, 64)],
                                      sem_ring).wait()
                return 0

            lax.fori_loop(0, nhp_v[...][0], _dr, 0)
            pltpu.make_async_copy(memT_hbm.at[:, pl.ds(0, LWA)],
                                  slab_v.at[pl.ds(0, 64), :], sem_in).wait()

            def _chunk(t, nhc):
                s16 = stv_v[pl.ds(t * 16, 16)]
                cnt_t = plsc.all_reduce_population_count(s16 >= 0)[0]

                @pl.when(cnt_t > 0)
                def _(par=par):
                    for l in range(16):
                        e = s16[l]

                        @pl.when(e >= 0)
                        def _(l=l):
                            p = t * 16 + l
                            for c4 in range(4):
                                val = plsc.load_gather(
                                    slab_v,
                                    [_splat(par * 64 + c4 * 16) + iota,
                                     _splat(p)])
                                plsc.store_scatter(
                                    ring_v,
                                    [_splat(p * 64 + c4 * 16) + iota], val)
                            pltpu.async_copy(
                                ring_v.at[pl.ds(p * 64, 64)],
                                srcmem_hbm.at[pl.ds(e * 64, 64)], sem_ring)
                return nhc + cnt_t

            nh = lax.fori_loop(0, LWA // 16, _chunk, 0)
            nhp_v[...] = _splat(nh)
            dexp.wait()
        return 0

    lax.fori_loop(0, SPTA, _super, 0)

    def _dr_last(k, _):
        pltpu.make_async_copy(srcmem_hbm.at[pl.ds(0, 64)],
                              ring_v.at[pl.ds(0, 64)], sem_ring).wait()
        return 0

    lax.fori_loop(0, nhp_v[...][0], _dr_last, 0)

    @pl.when(wid == NW - 1)
    def _tail():
        coff = NG * 128
        pltpu.async_copy(tail_hbm.at[:, :],
                         slab_v.at[pl.ds(0, 64), pl.ds(0, 128)], sem_in).wait()
        pltpu.sync_copy(stamp_sh.at[pl.ds(coff, 64)], stv_v.at[pl.ds(0, 64)])
        pltpu.async_copy(stv_v.at[pl.ds(0, 64)],
                         stamp_hbm.at[pl.ds(coff, 64)], sem_misc).wait()
        def _chunk_t(t, nhc):
            s16 = stv_v[pl.ds(t * 16, 16)]
            cnt_t = plsc.all_reduce_population_count(s16 >= 0)[0]

            @pl.when(cnt_t > 0)
            def _():
                for l in range(16):
                    e = s16[l]

                    @pl.when(e >= 0)
                    def _(l=l):
                        p = t * 16 + l
                        for c4 in range(4):
                            val = plsc.load_gather(
                                slab_v, [_splat(c4 * 16) + iota, _splat(p)])
                            plsc.store_scatter(
                                ring_v, [_splat(p * 64 + c4 * 16) + iota], val)
                        pltpu.async_copy(ring_v.at[pl.ds(p * 64, 64)],
                                         srcmem_hbm.at[pl.ds(e * 64, 64)],
                                         sem_ring)
            return nhc + cnt_t

        nh = lax.fori_loop(0, 4, _chunk_t, 0)

        def _dr(k, _):
            pltpu.make_async_copy(srcmem_hbm.at[pl.ds(0, 64)],
                                  ring_v.at[pl.ds(0, 64)], sem_ring).wait()
            return 0

        lax.fori_loop(0, nh, _dr, 0)


_R = 2048


def _phase_b_body(sm_ref, ef_ref, et_ref, sl_ref,
                  w1m_ref, w1e_ref, w1d_ref, b1_ref, w2_ref, b2_ref,
                  wih_ref, whh_ref, bih_ref, bhh_ref, out_ref):
    f32 = jnp.float32
    sm = sm_ref[...]
    ef = ef_ref[...]
    dt = et_ref[...] - sl_ref[...]
    x1 = (jnp.dot(sm, w1m_ref[...], preferred_element_type=f32)
          + jnp.dot(ef, w1e_ref[...], preferred_element_type=f32)
          + dt * w1d_ref[...] + b1_ref[...])
    h1 = jnp.maximum(x1, 0.0)
    msg = jnp.dot(h1, w2_ref[...], preferred_element_type=f32) + b2_ref[...]
    gi = jnp.dot(msg, wih_ref[...], preferred_element_type=f32) + bih_ref[...]
    gh = jnp.dot(sm, whh_ref[...], preferred_element_type=f32) + bhh_ref[...]
    r = jax.nn.sigmoid(gi[:, 0:MEM] + gh[:, 0:MEM])
    z = jax.nn.sigmoid(gi[:, MEM:2 * MEM] + gh[:, MEM:2 * MEM])
    n = jnp.tanh(gi[:, 2 * MEM:3 * MEM] + r * gh[:, 2 * MEM:3 * MEM])
    upd = (1.0 - z) * n + z * sm
    out_ref[...] = jnp.concatenate([upd, jnp.zeros((_R, 128 - MEM), f32)],
                                   axis=1)


def _phase_b(src_mem, ef, et2d, sl2d, w1m, w1e, w1d, b1, w2, b2,
             wih, whh, bih, bhh):
    full = lambda shape: pl.BlockSpec(shape, lambda i: (0, 0))
    return pl.pallas_call(
        _phase_b_body,
        grid=(B // _R,),
        in_specs=[
            pl.BlockSpec((_R, MEM), lambda i: (i, 0)),
            pl.BlockSpec((_R, MEM), lambda i: (i, 0)),
            pl.BlockSpec((_R, 1), lambda i: (i, 0)),
            pl.BlockSpec((_R, 1), lambda i: (i, 0)),
            full((MEM, MSG)), full((MEM, MSG)), full((1, MSG)), full((1, MSG)),
            full((MSG, MSG)), full((1, MSG)),
            full((MSG, 3 * MEM)), full((MEM, 3 * MEM)),
            full((1, 3 * MEM)), full((1, 3 * MEM)),
        ],
        out_specs=pl.BlockSpec((_R, 128), lambda i: (i, 0)),
        out_shape=jax.ShapeDtypeStruct((B, 128), jnp.float32),
    )(src_mem, ef, et2d, sl2d, w1m, w1e, w1d, b1, w2, b2, wih, whh, bih, bhh)


SG = 4
LW = 128 * SG
NSUP = (NG * 128) // LW
SPT = 62


@functools.partial(
    pl.kernel,
    out_type=jax.ShapeDtypeStruct((MEM, NPAD), jnp.float32),
    mesh=_mesh,
    compiler_params=pltpu.CompilerParams(needs_layout_passes=False),
    scratch_types=[
        pltpu.VMEM((128, LW), jnp.float32),
        pltpu.VMEM((2 * LW,), jnp.int32),
        pltpu.VMEM((KUP * 8, 128), jnp.float32),
        pltpu.SemaphoreType.DMA,
        pltpu.SemaphoreType.DMA,
        pltpu.SemaphoreType.DMA,
        pltpu.SemaphoreType.DMA,
    ],
)
def _phase_c(memT_hbm, upd_hbm, stamp_hbm, tail_hbm, outT_hbm,
             slab_v, stv_v, updr_v,
             sem_in, sem_out, sem_up, sem_st):
    cid = lax.axis_index("c")
    sid = lax.axis_index("s")
    wid = cid * NT + sid
    iota = _i16()
    s0 = wid * SPT
    V = jnp.clip(NSUP - s0, 0, SPT)

    @pl.when(V > 0)
    def _prologue():
        off0 = pl.multiple_of(s0 * LW, 128)
        pltpu.async_copy(memT_hbm.at[:, pl.ds(off0, LW)],
                         slab_v.at[pl.ds(0, 64), :], sem_in)
        pltpu.async_copy(stamp_hbm.at[pl.ds(off0, LW)],
                         stv_v.at[pl.ds(0, LW)], sem_st)

    def _super(gs, _):
        @pl.when(gs < V)
        def _():
            par = gs % 2
            g = s0 + gs

            @pl.when((gs >= 1) & (gs + 1 < V))
            def _():
                pltpu.make_async_copy(memT_hbm.at[:, pl.ds(0, LW)],
                                      slab_v.at[pl.ds(0, 64), :],
                                      sem_out).wait()

            @pl.when(gs + 1 < V)
            def _():
                noff = pl.multiple_of((g + 1) * LW, 128)
                pltpu.async_copy(memT_hbm.at[:, pl.ds(noff, LW)],
                                 slab_v.at[pl.ds((1 - par) * 64, 64), :],
                                 sem_in)
                pltpu.async_copy(stamp_hbm.at[pl.ds(noff, LW)],
                                 stv_v.at[pl.ds((1 - par) * LW, LW)], sem_st)
            pltpu.make_async_copy(stamp_hbm.at[pl.ds(0, LW)],
                                  stv_v.at[pl.ds(0, LW)], sem_st).wait()
            pltpu.make_async_copy(memT_hbm.at[:, pl.ds(0, LW)],
                                  slab_v.at[pl.ds(0, 64), :], sem_in).wait()
            sbase = par * LW

            def _cnt(t, a):
                s16 = stv_v[pl.ds(sbase + t * 16, 16)]
                return a + plsc.all_reduce_population_count(s16 >= 0)[0]

            nh = lax.fori_loop(0, LW // 16, _cnt, 0)

            @pl.when(nh > 0)
            def _(par=par, sbase=sbase, nh=nh):
                def _wave(w, _):
                    wlo = w * KUP
                    whi = wlo + KUP

                    def _chunk_issue(t, base):
                        s16 = stv_v[pl.ds(sbase + t * 16, 16)]
                        cnt = plsc.all_reduce_population_count(s16 >= 0)[0]

                        @pl.when((cnt > 0) & (base < whi)
                                 & (base + cnt > wlo))
                        def _():
                            lc = base
                            for l in range(16):
                                e = s16[l]
                                onr = e >= 0

                                @pl.when(onr & (lc >= wlo) & (lc < whi))
                                def _(e=e, lc=lc):
                                    e8 = pl.multiple_of((e // 8) * 8, 8)
                                    pltpu.async_copy(
                                        upd_hbm.at[pl.ds(e8, 8), :],
                                        updr_v.at[pl.ds((lc - wlo) * 8, 8), :],
                                        sem_up)
                                lc = lc + jnp.where(onr, 1, 0)
                        return base + cnt

                    lax.fori_loop(0, LW // 16, _chunk_issue, 0)
                    cw = jnp.minimum(nh, whi) - wlo

                    def _drw(k, _):
                        pltpu.make_async_copy(upd_hbm.at[pl.ds(0, 8), :],
                                              updr_v.at[pl.ds(0, 8), :],
                                              sem_up).wait()
                        return 0

                    lax.fori_loop(0, cw, _drw, 0)

                    def _chunk_apply(t, base):
                        s16 = stv_v[pl.ds(sbase + t * 16, 16)]
                        cnt = plsc.all_reduce_population_count(s16 >= 0)[0]

                        @pl.when((cnt > 0) & (base < whi)
                                 & (base + cnt > wlo))
                        def _():
                            lc = base
                            for l in range(16):
                                e = s16[l]
                                onr = e >= 0

                                @pl.when(onr & (lc >= wlo) & (lc < whi))
                                def _(e=e, lc=lc, l=l):
                                    r = (lc - wlo) * 8 + (e - (e // 8) * 8)
                                    p = t * 16 + l
                                    for c4 in range(4):
                                        val = plsc.load_gather(
                                            updr_v,
                                            [_splat(r), c4 * 16 + iota])
                                        plsc.store_scatter(
                                            slab_v,
                                            [_splat(par * 64 + c4 * 16) + iota,
                                             _splat(p)], val)
                                lc = lc + jnp.where(onr, 1, 0)
                        return base + cnt

                    lax.fori_loop(0, LW // 16, _chunk_apply, 0)
                    return 0

                lax.fori_loop(0, (nh + KUP - 1) // KUP, _wave, 0)

            coff = pl.multiple_of(g * LW, 128)
            pltpu.async_copy(slab_v.at[pl.ds(par * 64, 64), :],
                             outT_hbm.at[:, pl.ds(coff, LW)], sem_out)
        return 0

    lax.fori_loop(0, SPT, _super, 0)

    @pl.when(V >= 2)
    def _ep1():
        pltpu.make_async_copy(memT_hbm.at[:, pl.ds(0, LW)],
                              slab_v.at[pl.ds(0, 64), :], sem_out).wait()

    @pl.when(V >= 1)
    def _ep2():
        pltpu.make_async_copy(memT_hbm.at[:, pl.ds(0, LW)],
                              slab_v.at[pl.ds(0, 64), :], sem_out).wait()

    @pl.when(wid == NW - 1)
    def _tail():
        coff = NG * 128
        pltpu.async_copy(tail_hbm.at[:, :],
                         slab_v.at[pl.ds(0, 64), pl.ds(0, 128)], sem_in).wait()
        pltpu.sync_copy(stamp_hbm.at[pl.ds(coff, 64)], stv_v.at[pl.ds(0, 64)])
        for t in range(4):
            s16 = stv_v[pl.ds(t * 16, 16)]
            for l in range(16):
                e = s16[l]

                @pl.when(e >= 0)
                def _(e=e, t=t, l=l):
                    e8 = pl.multiple_of((e // 8) * 8, 8)
                    pltpu.async_copy(upd_hbm.at[pl.ds(e8, 8), :],
                                     updr_v.at[pl.ds(0, 8), :], sem_up).wait()
                    r = e - (e // 8) * 8
                    p = t * 16 + l
                    for c4 in range(4):
                        val = plsc.load_gather(updr_v,
                                               [_splat(r), c4 * 16 + iota])
                        plsc.store_scatter(slab_v,
                                           [c4 * 16 + iota, _splat(p)], val)
        pltpu.async_copy(slab_v.at[pl.ds(0, 64), pl.ds(0, 128)],
                         outT_hbm.at[:, pl.ds(coff, 128)], sem_out).wait()


def kernel(source_nodes, edge_times, edge_features, memory, last_update,
           W1, b1, W2, b2, W_ih, W_hh, b_ih, b_hh):
    idx2d = source_nodes.reshape(B // 128, 128)
    memT = memory.T
    tail_in = jnp.pad(memory[NG * 128:, :].T, ((0, 0), (0, NPAD - N)))
    srcmem_lin, src_last2d, stamp = _phase_a(memT, last_update, idx2d, tail_in)
    updated = _phase_b(
        srcmem_lin.reshape(B, MEM), edge_features,
        edge_times.reshape(B, 1), src_last2d.reshape(B, 1),
        W1[:, :MEM].T, W1[:, MEM:2 * MEM].T, W1[:, 2 * MEM].reshape(1, MSG),
        b1.reshape(1, MSG), W2.T, b2.reshape(1, MSG),
        W_ih.T, W_hh.T, b_ih.reshape(1, 3 * MEM), b_hh.reshape(1, 3 * MEM),
    )
    outT = _phase_c(memT, updated, stamp, tail_in)
    return outT[:, :N].T
